# Initial kernel scaffold; baseline (speedup 1.0000x reference)
import jax
import jax.numpy as jnp
from jax.experimental import pallas as pl


def kernel(x, W, b):
    raise NotImplementedError("write your pallas kernel here")


# speedup vs baseline: 1.0797x; 1.0797x over previous
import jax
import jax.numpy as jnp
from jax.experimental import pallas as pl
from jax.experimental.pallas import tpu as pltpu

F_BLK = 256


def _gate_body(x_ref, w_ref, b_ref, gate_ref, mask_ref, acc_ref):
    i = pl.program_id(0)
    xb = x_ref[...]
    s = jnp.sum(xb, axis=-1) * (1.0 / 2048.0) + jnp.max(xb, axis=-1)
    hp = jax.lax.dot_general(
        s, w_ref[...], (((1,), (1,)), ((), ())),
        preferred_element_type=jnp.float32,
    )

    @pl.when(i == 0)
    def _init():
        acc_ref[...] = hp + b_ref[...][None, :]

    @pl.when(i > 0)
    def _accum():
        acc_ref[...] = acc_ref[...] + hp

    @pl.when(i == pl.num_programs(0) - 1)
    def _epilogue():
        h = acc_ref[...]
        h = jnp.where(h >= 0.0, h, 0.2 * h)
        iota = jax.lax.broadcasted_iota(jnp.int32, h.shape, 1)
        m1 = jnp.max(h, axis=1, keepdims=True)
        i1 = jnp.min(jnp.where(h == m1, iota, 16), axis=1, keepdims=True)
        h2 = jnp.where(iota == i1, -jnp.inf, h)
        m2 = jnp.max(h2, axis=1, keepdims=True)
        i2 = jnp.min(jnp.where(h2 == m2, iota, 16), axis=1, keepdims=True)
        sel = (iota == i1) | (iota == i2)
        mask_ref[...] = sel.astype(jnp.float32)
        d = jnp.where(sel, jnp.exp(h - m1), 0.0)
        gate_ref[...] = d / jnp.sum(d, axis=1, keepdims=True)


def kernel(x, W, b):
    B, F, C = x.shape
    E = W.shape[0]
    grid = (F // F_BLK,)
    gating, mask = pl.pallas_call(
        _gate_body,
        grid=grid,
        in_specs=[
            pl.BlockSpec((B, F_BLK, C), lambda i: (0, i, 0)),
            pl.BlockSpec((E, F_BLK), lambda i: (0, i)),
            pl.BlockSpec((E,), lambda i: (0,)),
        ],
        out_specs=[
            pl.BlockSpec((B, E), lambda i: (0, 0)),
            pl.BlockSpec((B, E), lambda i: (0, 0)),
        ],
        out_shape=[
            jax.ShapeDtypeStruct((B, E), jnp.float32),
            jax.ShapeDtypeStruct((B, E), jnp.float32),
        ],
        scratch_shapes=[pltpu.VMEM((B, E), jnp.float32)],
    )(x, W, b)
    return gating, mask
